# baseline (speedup 1.0000x reference)
import jax
import jax.numpy as jnp
from jax.experimental import pallas as pl
from jax.experimental.pallas import tpu as pltpu

_MIN_NORM = 1e-15
_BALL_EPS = 1e-5


def _rownorm(v):
    return jnp.maximum(jnp.sqrt(jnp.sum(v * v, axis=-1, keepdims=True)), _MIN_NORM)


def _support_body(x_ref, w_ref, b_ref, out_ref):
    xv = x_ref[...]
    maxnorm = 1.0 - _BALL_EPS
    norm = _rownorm(xv)
    xp = jnp.where(norm > maxnorm, xv / norm * maxnorm, xv)
    n2 = _rownorm(xp)
    v = jnp.clip(n2, -1.0 + 1e-7, 1.0 - 1e-7)
    at = 0.5 * (jnp.log1p(v) - jnp.log1p(-v))
    xt = xp / n2 * at
    out_ref[...] = (
        jax.lax.dot_general(
            xt, w_ref[...], (((1,), (0,)), ((), ())),
            preferred_element_type=jnp.float32,
        )
        + b_ref[...]
    )


def _agg_body(adj_ref, s_ref, out_ref):
    acc = jax.lax.dot_general(
        adj_ref[...], s_ref[...], (((1,), (0,)), ((), ())),
        preferred_element_type=jnp.float32,
    )
    norm = _rownorm(acc)
    gamma = jnp.tanh(norm) * acc / norm
    gnorm = _rownorm(gamma)
    maxnorm = 1.0 - _BALL_EPS
    out_ref[...] = jnp.where(gnorm > maxnorm, gamma / gnorm * maxnorm, gamma)


def kernel(x, adj, weight, bias):
    n, d_in = x.shape
    d_out = weight.shape[1]
    bias2 = bias.reshape(1, d_out).astype(jnp.float32)

    bn = 1000 if n % 1000 == 0 else n
    support = pl.pallas_call(
        _support_body,
        grid=(n // bn,),
        in_specs=[
            pl.BlockSpec((bn, d_in), lambda i: (i, 0)),
            pl.BlockSpec((d_in, d_out), lambda i: (0, 0)),
            pl.BlockSpec((1, d_out), lambda i: (0, 0)),
        ],
        out_specs=pl.BlockSpec((bn, d_out), lambda i: (i, 0)),
        out_shape=jax.ShapeDtypeStruct((n, d_out), jnp.float32),
        compiler_params=pltpu.CompilerParams(
            dimension_semantics=("parallel",)),
    )(x, weight, bias2)

    bm = 80 if n % 80 == 0 else n
    out = pl.pallas_call(
        _agg_body,
        grid=(n // bm,),
        in_specs=[
            pl.BlockSpec((bm, n), lambda i: (i, 0)),
            pl.BlockSpec((n, d_out), lambda i: (0, 0)),
        ],
        out_specs=pl.BlockSpec((bm, d_out), lambda i: (i, 0)),
        out_shape=jax.ShapeDtypeStruct((n, d_out), jnp.float32),
        compiler_params=pltpu.CompilerParams(
            dimension_semantics=("parallel",)),
    )(adj, support)
    return out


# speedup vs baseline: 1.4082x; 1.4082x over previous
import jax
import jax.numpy as jnp
from jax.experimental import pallas as pl
from jax.experimental.pallas import tpu as pltpu

_MIN_NORM = 1e-15
_BALL_EPS = 1e-5


def _rownorm(v):
    return jnp.maximum(jnp.sqrt(jnp.sum(v * v, axis=-1, keepdims=True)), _MIN_NORM)


def _support_body(x_ref, w_ref, b_ref, out_ref):
    xv = x_ref[...]
    maxnorm = 1.0 - _BALL_EPS
    norm = _rownorm(xv)
    xp = jnp.where(norm > maxnorm, xv / norm * maxnorm, xv)
    n2 = _rownorm(xp)
    v = jnp.clip(n2, -1.0 + 1e-7, 1.0 - 1e-7)
    at = 0.5 * (jnp.log1p(v) - jnp.log1p(-v))
    xt = xp / n2 * at
    out_ref[...] = (
        jax.lax.dot_general(
            xt, w_ref[...], (((1,), (0,)), ((), ())),
            preferred_element_type=jnp.float32,
        )
        + b_ref[...]
    )


def _agg_body(adj_ref, s_ref, out_ref):
    acc = jax.lax.dot_general(
        adj_ref[...], s_ref[...], (((1,), (0,)), ((), ())),
        preferred_element_type=jnp.float32,
    )
    norm = _rownorm(acc)
    gamma = jnp.tanh(norm) * acc / norm
    gnorm = _rownorm(gamma)
    maxnorm = 1.0 - _BALL_EPS
    out_ref[...] = jnp.where(gnorm > maxnorm, gamma / gnorm * maxnorm, gamma)


def kernel(x, adj, weight, bias):
    n, d_in = x.shape
    d_out = weight.shape[1]
    bias2 = bias.reshape(1, d_out).astype(jnp.float32)

    bn = 1000 if n % 1000 == 0 else n
    support = pl.pallas_call(
        _support_body,
        grid=(n // bn,),
        in_specs=[
            pl.BlockSpec((bn, d_in), lambda i: (i, 0)),
            pl.BlockSpec((d_in, d_out), lambda i: (0, 0)),
            pl.BlockSpec((1, d_out), lambda i: (0, 0)),
        ],
        out_specs=pl.BlockSpec((bn, d_out), lambda i: (i, 0)),
        out_shape=jax.ShapeDtypeStruct((n, d_out), jnp.float32),
        compiler_params=pltpu.CompilerParams(
            dimension_semantics=("parallel",)),
    )(x, weight, bias2)

    bm = 400 if n % 400 == 0 else n
    out = pl.pallas_call(
        _agg_body,
        grid=(n // bm,),
        in_specs=[
            pl.BlockSpec((bm, n), lambda i: (i, 0)),
            pl.BlockSpec((n, d_out), lambda i: (0, 0)),
        ],
        out_specs=pl.BlockSpec((bm, d_out), lambda i: (i, 0)),
        out_shape=jax.ShapeDtypeStruct((n, d_out), jnp.float32),
        compiler_params=pltpu.CompilerParams(
            dimension_semantics=("parallel",)),
    )(adj, support)
    return out
